# baseline (speedup 1.0000x reference)
import functools

import jax
import jax.numpy as jnp
from jax import lax
from jax.experimental import pallas as pl
from jax.experimental.pallas import tpu as pltpu
from jax.experimental.pallas import tpu_sc as plsc

_NC = 2
_NS = 16
_NW = _NC * _NS
_L = 16

_N = 320000
_D = 128
_DE = 16
_R = 256
_CHUNKS = _N // _R


def _sc_body(src_hbm, et_hbm, tgt_hbm, out_hbm, src_v, tgt_v, e_v):
    wid = lax.axis_index("s") * _NC + lax.axis_index("c")
    n_mine = (_CHUNKS - wid + _NW - 1) // _NW

    def chunk(k, carry):
        base = (wid + k * _NW) * _R
        pltpu.sync_copy(src_hbm.at[pl.ds(base, _R), :], src_v)
        pltpu.sync_copy(tgt_hbm.at[pl.ds(base, _R), :], tgt_v)
        pltpu.sync_copy(et_hbm.at[:, pl.ds(base, _R)], e_v)

        def grp(j, carry2):
            acc = e_v[0, pl.ds(j * _L, _L)]
            for kk in range(1, _DE):
                acc = acc + e_v[kk, pl.ds(j * _L, _L)]
            g16 = 1.0 / (1.0 + jnp.exp(acc * (-1.0 / _DE)))
            row0 = j * _L
            for t in range(_L):
                g = jnp.take(g16, jnp.full((_L,), t, jnp.int32))
                for cc in range(_D // _L):
                    sl = pl.ds(cc * _L, _L)
                    src_v[row0 + t, sl] = src_v[row0 + t, sl] * g + tgt_v[row0 + t, sl]
            return carry2

        lax.fori_loop(0, _R // _L, grp, 0)

        pltpu.sync_copy(src_v, out_hbm.at[pl.ds(base, _R), :])
        return carry

    lax.fori_loop(0, n_mine, chunk, 0)


def kernel(src, e, tgt):
    mesh = plsc.VectorSubcoreMesh(core_axis_name="c", subcore_axis_name="s")
    f = functools.partial(
        pl.kernel,
        out_type=jax.ShapeDtypeStruct((_N, _D), jnp.float32),
        mesh=mesh,
        scratch_types=[
            pltpu.VMEM((_R, _D), jnp.float32),
            pltpu.VMEM((_R, _D), jnp.float32),
            pltpu.VMEM((_DE, _R), jnp.float32),
        ],
    )(_sc_body)
    return f(src, e.T, tgt)
